# baseline (speedup 1.0000x reference)
import jax
import jax.numpy as jnp
from jax.experimental import pallas as pl
from jax.experimental.pallas import tpu as pltpu

IN_F = 1024
OUT_F = 1024
KSEL = 16
BATCH = 256
CHUNKS = IN_F // KSEL


def _oddeven_pairs(n):
    sort_pairs = []
    merge_pairs = []

    def merge(lo, m, r, out):
        step = r * 2
        if step < m:
            merge(lo, m, step, out)
            merge(lo + r, m, step, out)
            for i in range(lo + r, lo + m - r, step):
                out.append((i, i + r))
        else:
            out.append((lo, lo + r))

    def sort(lo, m):
        if m > 1:
            half = m // 2
            sort(lo, half)
            sort(lo + half, half)
            merge(lo, m, 1, sort_pairs)

    sort(0, n)
    merge(0, 2 * n, 1, merge_pairs)
    return sort_pairs, merge_pairs


_MERGE32 = _oddeven_pairs(KSEL)[1]

_SORT16 = [
    (0, 1), (2, 3), (4, 5), (6, 7), (8, 9), (10, 11), (12, 13), (14, 15),
    (0, 2), (4, 6), (8, 10), (12, 14), (1, 3), (5, 7), (9, 11), (13, 15),
    (0, 4), (8, 12), (1, 5), (9, 13), (2, 6), (10, 14), (3, 7), (11, 15),
    (0, 8), (1, 9), (2, 10), (3, 11), (4, 12), (5, 13), (6, 14), (7, 15),
    (5, 10), (6, 9), (3, 12), (13, 14), (7, 11), (1, 2), (4, 8),
    (1, 4), (7, 13), (2, 8), (11, 14), (5, 6), (9, 10),
    (2, 4), (11, 13), (3, 8), (7, 12),
    (6, 8), (10, 12), (3, 5), (7, 9),
    (3, 4), (5, 6), (7, 8), (9, 10), (11, 12),
    (6, 7), (8, 9),
]


def _ce_desc(planes, pairs):
    planes = list(planes)
    for i, j in pairs:
        a, b = planes[i], planes[j]
        planes[i] = jnp.maximum(a, b)
        planes[j] = jnp.minimum(a, b)
    return planes


def _bitonic_merge(planes, descending):
    planes = list(planes)
    for d in (8, 4, 2, 1):
        for i in range(KSEL):
            if i & d == 0:
                a, b = planes[i], planes[i + d]
                if descending:
                    planes[i] = jnp.maximum(a, b)
                    planes[i + d] = jnp.minimum(a, b)
                else:
                    planes[i] = jnp.minimum(a, b)
                    planes[i + d] = jnp.maximum(a, b)
    return planes


def _tree_sum(planes):
    vals = list(planes)
    while len(vals) > 1:
        nxt = [vals[i] + vals[i + 1] for i in range(0, len(vals) - 1, 2)]
        if len(vals) % 2:
            nxt.append(vals[-1])
        vals = nxt
    return vals[0]


def _mm_body(x_ref, w2_ref, out_ref):
    out_ref[...] = jax.lax.dot_general(
        x_ref[...], w2_ref[...], (((1,), (0,)), ((), ())),
        preferred_element_type=jnp.float32)


def _make_run(x_ref, wt_ref, base):
    w32 = wt_ref[pl.ds(base, 2 * KSEL), :, :]
    planes = [x_ref[0, 0, base + p] * w32[p] for p in range(2 * KSEL)]
    sa = _ce_desc(planes[:KSEL], _SORT16)
    sb = _ce_desc(planes[KSEL:], _SORT16)
    return _ce_desc(sa + sb, _MERGE32)


def _merge_top(a, b):
    c = [jnp.maximum(a[p], b[KSEL - 1 - p]) for p in range(KSEL)]
    return _bitonic_merge(c, descending=True)


def _merge_bot(a, b):
    c = [jnp.minimum(a[p], b[KSEL - 1 - p]) for p in range(KSEL)]
    return _bitonic_merge(c, descending=False)


def _block(x_ref, wt_ref, base):
    r1 = _make_run(x_ref, wt_ref, base)
    r2 = _make_run(x_ref, wt_ref, base + 2 * KSEL)
    ptop = _merge_top(r1[:KSEL], r2[:KSEL])
    pbot = _merge_bot(r1[:KSEL - 1:-1], r2[:KSEL - 1:-1])
    r3 = _make_run(x_ref, wt_ref, base + 4 * KSEL)
    r4 = _make_run(x_ref, wt_ref, base + 6 * KSEL)
    qtop = _merge_top(r3[:KSEL], r4[:KSEL])
    qbot = _merge_bot(r3[:KSEL - 1:-1], r4[:KSEL - 1:-1])
    return _merge_top(ptop, qtop), _merge_bot(pbot, qbot)


def _body(x_ref, wt_ref, out_ref):
    nblk = CHUNKS // 8

    top, bot = _block(x_ref, wt_ref, 0)

    def step(i, carry):
        top, bot = carry
        ptop, pbot = _block(x_ref, wt_ref, 8 * KSEL * i)
        top = tuple(_merge_top(list(top), ptop))
        bot = tuple(_merge_bot(list(bot), pbot))
        return top, bot

    top, bot = jax.lax.fori_loop(1, nblk - 1, step,
                                 (tuple(top), tuple(bot)))

    ptop, pbot = _block(x_ref, wt_ref, 8 * KSEL * (nblk - 1))
    tsum = _tree_sum([jnp.maximum(top[p], ptop[KSEL - 1 - p])
                      for p in range(KSEL)])
    bsum = _tree_sum([jnp.minimum(bot[p], pbot[KSEL - 1 - p])
                      for p in range(KSEL)])
    out_ref[0] = tsum - bsum


@jax.jit
def kernel(x, W):
    wt2 = W.T
    wt3 = wt2.reshape(IN_F, 8, 128)
    x3 = x.reshape(BATCH, 1, IN_F)
    mm = pl.pallas_call(
        _mm_body,
        grid=(4,),
        in_specs=[
            pl.BlockSpec((BATCH // 4, IN_F), lambda r: (r, 0)),
            pl.BlockSpec((IN_F, OUT_F), lambda r: (0, 0)),
        ],
        out_specs=pl.BlockSpec((BATCH // 4, OUT_F), lambda r: (r, 0)),
        out_shape=jax.ShapeDtypeStruct((BATCH, OUT_F), jnp.float32),
    )(x, wt2)
    corr3 = pl.pallas_call(
        _body,
        grid=(BATCH,),
        in_specs=[
            pl.BlockSpec((1, 1, IN_F), lambda b: (b, 0, 0),
                         memory_space=pltpu.SMEM),
            pl.BlockSpec((IN_F, 8, 128), lambda b: (0, 0, 0)),
        ],
        out_specs=pl.BlockSpec((1, 8, 128), lambda b: (b, 0, 0)),
        out_shape=jax.ShapeDtypeStruct((BATCH, 8, 128), jnp.float32),
    )(x3, wt3)
    return mm - corr3.reshape(BATCH, OUT_F)


# speedup vs baseline: 1.0341x; 1.0341x over previous
import jax
import jax.numpy as jnp
from jax.experimental import pallas as pl
from jax.experimental.pallas import tpu as pltpu

IN_F = 1024
OUT_F = 1024
KSEL = 16
BATCH = 256
CHUNKS = IN_F // KSEL


def _oddeven_pairs(n):
    sort_pairs = []
    merge_pairs = []

    def merge(lo, m, r, out):
        step = r * 2
        if step < m:
            merge(lo, m, step, out)
            merge(lo + r, m, step, out)
            for i in range(lo + r, lo + m - r, step):
                out.append((i, i + r))
        else:
            out.append((lo, lo + r))

    def sort(lo, m):
        if m > 1:
            half = m // 2
            sort(lo, half)
            sort(lo + half, half)
            merge(lo, m, 1, sort_pairs)

    sort(0, n)
    merge(0, 2 * n, 1, merge_pairs)
    return sort_pairs, merge_pairs


_MERGE32 = _oddeven_pairs(KSEL)[1]

_SORT16 = [
    (0, 1), (2, 3), (4, 5), (6, 7), (8, 9), (10, 11), (12, 13), (14, 15),
    (0, 2), (4, 6), (8, 10), (12, 14), (1, 3), (5, 7), (9, 11), (13, 15),
    (0, 4), (8, 12), (1, 5), (9, 13), (2, 6), (10, 14), (3, 7), (11, 15),
    (0, 8), (1, 9), (2, 10), (3, 11), (4, 12), (5, 13), (6, 14), (7, 15),
    (5, 10), (6, 9), (3, 12), (13, 14), (7, 11), (1, 2), (4, 8),
    (1, 4), (7, 13), (2, 8), (11, 14), (5, 6), (9, 10),
    (2, 4), (11, 13), (3, 8), (7, 12),
    (6, 8), (10, 12), (3, 5), (7, 9),
    (3, 4), (5, 6), (7, 8), (9, 10), (11, 12),
    (6, 7), (8, 9),
]


def _ce_desc(planes, pairs):
    planes = list(planes)
    for i, j in pairs:
        a, b = planes[i], planes[j]
        planes[i] = jnp.maximum(a, b)
        planes[j] = jnp.minimum(a, b)
    return planes


def _bitonic_merge(planes, descending):
    planes = list(planes)
    for d in (8, 4, 2, 1):
        for i in range(KSEL):
            if i & d == 0:
                a, b = planes[i], planes[i + d]
                if descending:
                    planes[i] = jnp.maximum(a, b)
                    planes[i + d] = jnp.minimum(a, b)
                else:
                    planes[i] = jnp.minimum(a, b)
                    planes[i + d] = jnp.maximum(a, b)
    return planes


def _tree_sum(planes):
    vals = list(planes)
    while len(vals) > 1:
        nxt = [vals[i] + vals[i + 1] for i in range(0, len(vals) - 1, 2)]
        if len(vals) % 2:
            nxt.append(vals[-1])
        vals = nxt
    return vals[0]


def _mm_body(x_ref, w2_ref, out_ref):
    out_ref[...] = jax.lax.dot_general(
        x_ref[...], w2_ref[...], (((1,), (0,)), ((), ())),
        preferred_element_type=jnp.float32)


def _make_run(x_ref, wt_ref, base):
    w32 = wt_ref[pl.ds(base, 2 * KSEL), :, :]
    planes = [x_ref[0, 0, base + p] * w32[p] for p in range(2 * KSEL)]
    sa = _ce_desc(planes[:KSEL], _SORT16)
    sb = _ce_desc(planes[KSEL:], _SORT16)
    return _ce_desc(sa + sb, _MERGE32)


def _merge_top(a, b):
    c = [jnp.maximum(a[p], b[KSEL - 1 - p]) for p in range(KSEL)]
    return _bitonic_merge(c, descending=True)


def _merge_bot(a, b):
    c = [jnp.minimum(a[p], b[KSEL - 1 - p]) for p in range(KSEL)]
    return _bitonic_merge(c, descending=False)


def _block(x_ref, wt_ref, base):
    r1 = _make_run(x_ref, wt_ref, base)
    r2 = _make_run(x_ref, wt_ref, base + 2 * KSEL)
    ptop = _merge_top(r1[:KSEL], r2[:KSEL])
    pbot = _merge_bot(r1[:KSEL - 1:-1], r2[:KSEL - 1:-1])
    r3 = _make_run(x_ref, wt_ref, base + 4 * KSEL)
    r4 = _make_run(x_ref, wt_ref, base + 6 * KSEL)
    qtop = _merge_top(r3[:KSEL], r4[:KSEL])
    qbot = _merge_bot(r3[:KSEL - 1:-1], r4[:KSEL - 1:-1])
    return _merge_top(ptop, qtop), _merge_bot(pbot, qbot)


def _body(x_ref, wt_ref, out_ref):
    nblk = CHUNKS // 8

    top, bot = _block(x_ref, wt_ref, 0)

    for i in range(1, nblk - 1):
        ptop, pbot = _block(x_ref, wt_ref, 8 * KSEL * i)
        top = _merge_top(top, ptop)
        bot = _merge_bot(bot, pbot)

    ptop, pbot = _block(x_ref, wt_ref, 8 * KSEL * (nblk - 1))
    tsum = _tree_sum([jnp.maximum(top[p], ptop[KSEL - 1 - p])
                      for p in range(KSEL)])
    bsum = _tree_sum([jnp.minimum(bot[p], pbot[KSEL - 1 - p])
                      for p in range(KSEL)])
    out_ref[0] = tsum - bsum


@jax.jit
def kernel(x, W):
    wt2 = W.T
    wt3 = wt2.reshape(IN_F, 8, 128)
    x3 = x.reshape(BATCH, 1, IN_F)
    mm = pl.pallas_call(
        _mm_body,
        in_specs=[
            pl.BlockSpec((BATCH, IN_F), lambda: (0, 0)),
            pl.BlockSpec((IN_F, OUT_F), lambda: (0, 0)),
        ],
        out_specs=pl.BlockSpec((BATCH, OUT_F), lambda: (0, 0)),
        out_shape=jax.ShapeDtypeStruct((BATCH, OUT_F), jnp.float32),
    )(x, wt2)
    corr3 = pl.pallas_call(
        _body,
        grid=(BATCH,),
        in_specs=[
            pl.BlockSpec((1, 1, IN_F), lambda b: (b, 0, 0),
                         memory_space=pltpu.SMEM),
            pl.BlockSpec((IN_F, 8, 128), lambda b: (0, 0, 0)),
        ],
        out_specs=pl.BlockSpec((1, 8, 128), lambda b: (b, 0, 0)),
        out_shape=jax.ShapeDtypeStruct((BATCH, 8, 128), jnp.float32),
    )(x3, wt3)
    return mm - corr3.reshape(BATCH, OUT_F)


# speedup vs baseline: 1.0430x; 1.0086x over previous
import jax
import jax.numpy as jnp
from jax.experimental import pallas as pl
from jax.experimental.pallas import tpu as pltpu

IN_F = 1024
OUT_F = 1024
KSEL = 16
BATCH = 256
CHUNKS = IN_F // KSEL


def _oddeven_pairs(n):
    sort_pairs = []
    merge_pairs = []

    def merge(lo, m, r, out):
        step = r * 2
        if step < m:
            merge(lo, m, step, out)
            merge(lo + r, m, step, out)
            for i in range(lo + r, lo + m - r, step):
                out.append((i, i + r))
        else:
            out.append((lo, lo + r))

    def sort(lo, m):
        if m > 1:
            half = m // 2
            sort(lo, half)
            sort(lo + half, half)
            merge(lo, m, 1, sort_pairs)

    sort(0, n)
    merge(0, 2 * n, 1, merge_pairs)
    return sort_pairs, merge_pairs


_MERGE32 = _oddeven_pairs(KSEL)[1]

_SORT16 = [
    (0, 1), (2, 3), (4, 5), (6, 7), (8, 9), (10, 11), (12, 13), (14, 15),
    (0, 2), (4, 6), (8, 10), (12, 14), (1, 3), (5, 7), (9, 11), (13, 15),
    (0, 4), (8, 12), (1, 5), (9, 13), (2, 6), (10, 14), (3, 7), (11, 15),
    (0, 8), (1, 9), (2, 10), (3, 11), (4, 12), (5, 13), (6, 14), (7, 15),
    (5, 10), (6, 9), (3, 12), (13, 14), (7, 11), (1, 2), (4, 8),
    (1, 4), (7, 13), (2, 8), (11, 14), (5, 6), (9, 10),
    (2, 4), (11, 13), (3, 8), (7, 12),
    (6, 8), (10, 12), (3, 5), (7, 9),
    (3, 4), (5, 6), (7, 8), (9, 10), (11, 12),
    (6, 7), (8, 9),
]


def _ce_desc(planes, pairs):
    planes = list(planes)
    for i, j in pairs:
        a, b = planes[i], planes[j]
        planes[i] = jnp.maximum(a, b)
        planes[j] = jnp.minimum(a, b)
    return planes


def _bitonic_merge(planes, descending):
    planes = list(planes)
    for d in (8, 4, 2, 1):
        for i in range(KSEL):
            if i & d == 0:
                a, b = planes[i], planes[i + d]
                if descending:
                    planes[i] = jnp.maximum(a, b)
                    planes[i + d] = jnp.minimum(a, b)
                else:
                    planes[i] = jnp.minimum(a, b)
                    planes[i + d] = jnp.maximum(a, b)
    return planes


def _tree_sum(planes):
    vals = list(planes)
    while len(vals) > 1:
        nxt = [vals[i] + vals[i + 1] for i in range(0, len(vals) - 1, 2)]
        if len(vals) % 2:
            nxt.append(vals[-1])
        vals = nxt
    return vals[0]


def _mm_body(x_ref, w2_ref, out_ref):
    out_ref[...] = jax.lax.dot_general(
        x_ref[...], w2_ref[...], (((1,), (0,)), ((), ())),
        preferred_element_type=jnp.float32)


def _make_run(x_ref, wt_ref, row, base):
    w32 = wt_ref[pl.ds(base, 2 * KSEL), :, :]
    planes = [x_ref[0, row, base + p] * w32[p] for p in range(2 * KSEL)]
    sa = _ce_desc(planes[:KSEL], _SORT16)
    sb = _ce_desc(planes[KSEL:], _SORT16)
    return _ce_desc(sa + sb, _MERGE32)


def _merge_top(a, b):
    c = [jnp.maximum(a[p], b[KSEL - 1 - p]) for p in range(KSEL)]
    return _bitonic_merge(c, descending=True)


def _merge_bot(a, b):
    c = [jnp.minimum(a[p], b[KSEL - 1 - p]) for p in range(KSEL)]
    return _bitonic_merge(c, descending=False)


def _block(x_ref, wt_ref, row, base):
    r1 = _make_run(x_ref, wt_ref, row, base)
    r2 = _make_run(x_ref, wt_ref, row, base + 2 * KSEL)
    ptop = _merge_top(r1[:KSEL], r2[:KSEL])
    pbot = _merge_bot(r1[:KSEL - 1:-1], r2[:KSEL - 1:-1])
    r3 = _make_run(x_ref, wt_ref, row, base + 4 * KSEL)
    r4 = _make_run(x_ref, wt_ref, row, base + 6 * KSEL)
    qtop = _merge_top(r3[:KSEL], r4[:KSEL])
    qbot = _merge_bot(r3[:KSEL - 1:-1], r4[:KSEL - 1:-1])
    return _merge_top(ptop, qtop), _merge_bot(pbot, qbot)


def _row_corr(x_ref, wt_ref, row):
    nblk = CHUNKS // 8

    top, bot = _block(x_ref, wt_ref, row, 0)

    for i in range(1, nblk - 1):
        ptop, pbot = _block(x_ref, wt_ref, row, 8 * KSEL * i)
        top = _merge_top(top, ptop)
        bot = _merge_bot(bot, pbot)

    ptop, pbot = _block(x_ref, wt_ref, row, 8 * KSEL * (nblk - 1))
    tsum = _tree_sum([jnp.maximum(top[p], ptop[KSEL - 1 - p])
                      for p in range(KSEL)])
    bsum = _tree_sum([jnp.minimum(bot[p], pbot[KSEL - 1 - p])
                      for p in range(KSEL)])
    return tsum - bsum


def _body(x_ref, wt_ref, out_ref):
    out_ref[0, 0] = _row_corr(x_ref, wt_ref, 0)
    out_ref[0, 1] = _row_corr(x_ref, wt_ref, 1)


@jax.jit
def kernel(x, W):
    wt2 = W.T
    wt3 = wt2.reshape(IN_F, 8, 128)
    x3 = x.reshape(BATCH // 2, 2, IN_F)
    mm = pl.pallas_call(
        _mm_body,
        in_specs=[
            pl.BlockSpec((BATCH, IN_F), lambda: (0, 0)),
            pl.BlockSpec((IN_F, OUT_F), lambda: (0, 0)),
        ],
        out_specs=pl.BlockSpec((BATCH, OUT_F), lambda: (0, 0)),
        out_shape=jax.ShapeDtypeStruct((BATCH, OUT_F), jnp.float32),
    )(x, wt2)
    corr3 = pl.pallas_call(
        _body,
        grid=(BATCH // 2,),
        in_specs=[
            pl.BlockSpec((1, 2, IN_F), lambda b: (b, 0, 0),
                         memory_space=pltpu.SMEM),
            pl.BlockSpec((IN_F, 8, 128), lambda b: (0, 0, 0)),
        ],
        out_specs=pl.BlockSpec((1, 2, 8, 128), lambda b: (b, 0, 0, 0)),
        out_shape=jax.ShapeDtypeStruct((BATCH // 2, 2, 8, 128),
                                       jnp.float32),
    )(x3, wt3)
    return mm - corr3.reshape(BATCH, OUT_F)
